# baseline (speedup 1.0000x reference)
import functools

import jax
import jax.numpy as jnp
from jax.experimental import pallas as pl
from jax.experimental.pallas import tpu as pltpu


def _moe_body(xf_ref, xb_ref, gate_ref, wk_ref, bk_ref, wv_ref, bv_ref,
              wks_ref, bks_ref, wvs_ref, bvs_ref, out_ref, mask_ref):
    e = pl.program_id(0)
    hb = pl.program_id(1)
    x = xb_ref[...]

    @pl.when((e == 0) & (hb == 0))
    def _init():
        logits = jnp.dot(xf_ref[...], gate_ref[...],
                         preferred_element_type=jnp.float32)
        ids = jax.lax.broadcasted_iota(jnp.int32, logits.shape, 1)
        a1 = jnp.argmax(logits, axis=1, keepdims=True)
        s1 = jnp.max(logits, axis=1, keepdims=True)
        masked = jnp.where(ids == a1, -jnp.inf, logits)
        a2 = jnp.argmax(masked, axis=1, keepdims=True)
        s2 = jnp.max(masked, axis=1, keepdims=True)
        e2 = jnp.exp(s2 - s1)
        w1 = 1.0 / (1.0 + e2)
        w2 = e2 / (1.0 + e2)
        mask_ref[...] = (jnp.where(ids == a1, w1, 0.0)
                         + jnp.where(ids == a2, w2, 0.0))
        out_ref[...] = jnp.zeros_like(out_ref)

    ids = jax.lax.broadcasted_iota(jnp.int32, mask_ref.shape, 1)
    w_e = jnp.sum(mask_ref[...] * (ids == e), axis=1, keepdims=True)

    h = jax.nn.gelu(jnp.dot(x, wk_ref[0], preferred_element_type=jnp.float32)
                    + bk_ref[0])
    out_ref[...] += w_e * jnp.dot(h.astype(jnp.bfloat16), wv_ref[0],
                                  preferred_element_type=jnp.float32)

    @pl.when(hb == 0)
    def _bias():
        out_ref[...] += w_e * bv_ref[0]

    @pl.when(e == 0)
    def _shared():
        hs = jax.nn.gelu(jnp.dot(x, wks_ref[...],
                                 preferred_element_type=jnp.float32)
                         + bks_ref[...])
        out_ref[...] += jnp.dot(hs.astype(jnp.bfloat16), wvs_ref[...],
                                preferred_element_type=jnp.float32)

    @pl.when((e == 0) & (hb == 0))
    def _shared_bias():
        out_ref[...] += bvs_ref[...]


@functools.partial(jax.jit, static_argnames=())
def kernel(x, gate_kernel, Wk, bk, Wv, bv, Wk_s, bk_s, Wv_s, bv_s):
    B, S, D = x.shape
    T = B * S
    E = gate_kernel.shape[1]
    H = Wk.shape[2]
    HB = 512 if H % 512 == 0 else H
    NHB = H // HB

    x2 = x.reshape(T, D)
    xb = x2.astype(jnp.bfloat16)
    Wkb = Wk.astype(jnp.bfloat16)
    Wvb = Wv.astype(jnp.bfloat16)
    Wksb = Wk_s.astype(jnp.bfloat16)
    Wvsb = Wv_s.astype(jnp.bfloat16)
    bk2 = bk.reshape(E, 1, H)
    bv2 = bv.reshape(E, 1, D)
    bks2 = bk_s.reshape(1, H)
    bvs2 = bv_s.reshape(1, D)

    out = pl.pallas_call(
        _moe_body,
        grid=(E, NHB),
        in_specs=[
            pl.BlockSpec((T, D), lambda e, h: (0, 0)),
            pl.BlockSpec((T, D), lambda e, h: (0, 0)),
            pl.BlockSpec((D, E), lambda e, h: (0, 0)),
            pl.BlockSpec((1, D, HB), lambda e, h: (e, 0, h)),
            pl.BlockSpec((1, 1, HB), lambda e, h: (e, 0, h)),
            pl.BlockSpec((1, HB, D), lambda e, h: (e, h, 0)),
            pl.BlockSpec((1, 1, D), lambda e, h: (e, 0, 0)),
            pl.BlockSpec((D, HB), lambda e, h: (0, h)),
            pl.BlockSpec((1, HB), lambda e, h: (0, h)),
            pl.BlockSpec((HB, D), lambda e, h: (h, 0)),
            pl.BlockSpec((1, D), lambda e, h: (0, 0)),
        ],
        out_specs=pl.BlockSpec((T, D), lambda e, h: (0, 0)),
        out_shape=jax.ShapeDtypeStruct((T, D), jnp.float32),
        scratch_shapes=[pltpu.VMEM((T, E), jnp.float32)],
        compiler_params=pltpu.CompilerParams(
            dimension_semantics=("arbitrary", "arbitrary")),
    )(x2, xb, gate_kernel, Wkb, bk2, Wvb, bv2, Wksb, bks2, Wvsb, bvs2)

    return (out.reshape(B, S, D), jnp.float32(0.0))


# speedup vs baseline: 1.2637x; 1.2637x over previous
import functools

import jax
import jax.numpy as jnp
from jax.experimental import pallas as pl
from jax.experimental.pallas import tpu as pltpu


def _moe_body(xf_ref, xb_ref, gate_ref, wk_ref, bk_ref, wv_ref, bv_ref,
              wks_ref, bks_ref, wvs_ref, bvs_ref, out_ref, mask_ref):
    e = pl.program_id(0)
    hb = pl.program_id(1)
    x = xb_ref[...]
    wk = wk_ref[0].astype(jnp.bfloat16)
    wv = wv_ref[0].astype(jnp.bfloat16)

    @pl.when((e == 0) & (hb == 0))
    def _init():
        logits = jnp.dot(xf_ref[...], gate_ref[...],
                         preferred_element_type=jnp.float32)
        ids = jax.lax.broadcasted_iota(jnp.int32, logits.shape, 1)
        a1 = jnp.argmax(logits, axis=1, keepdims=True)
        s1 = jnp.max(logits, axis=1, keepdims=True)
        masked = jnp.where(ids == a1, -jnp.inf, logits)
        a2 = jnp.argmax(masked, axis=1, keepdims=True)
        s2 = jnp.max(masked, axis=1, keepdims=True)
        e2 = jnp.exp(s2 - s1)
        w1 = 1.0 / (1.0 + e2)
        w2 = e2 / (1.0 + e2)
        mask_ref[...] = (jnp.where(ids == a1, w1, 0.0)
                         + jnp.where(ids == a2, w2, 0.0))
        out_ref[...] = jnp.zeros_like(out_ref)

    ids = jax.lax.broadcasted_iota(jnp.int32, mask_ref.shape, 1)
    w_e = jnp.sum(mask_ref[...] * (ids == e), axis=1, keepdims=True)

    h = jax.nn.gelu(jnp.dot(x, wk, preferred_element_type=jnp.float32)
                    + bk_ref[0])
    out_ref[...] += w_e * jnp.dot(h.astype(jnp.bfloat16), wv,
                                  preferred_element_type=jnp.float32)

    @pl.when(hb == 0)
    def _bias():
        out_ref[...] += w_e * bv_ref[0]

    @pl.when(e == 0)
    def _shared():
        hs = jax.nn.gelu(jnp.dot(x, wks_ref[...].astype(jnp.bfloat16),
                                 preferred_element_type=jnp.float32)
                         + bks_ref[...])
        out_ref[...] += jnp.dot(hs.astype(jnp.bfloat16),
                                wvs_ref[...].astype(jnp.bfloat16),
                                preferred_element_type=jnp.float32)

    @pl.when((e == 0) & (hb == 0))
    def _shared_bias():
        out_ref[...] += bvs_ref[...]


@functools.partial(jax.jit, static_argnames=())
def kernel(x, gate_kernel, Wk, bk, Wv, bv, Wk_s, bk_s, Wv_s, bv_s):
    B, S, D = x.shape
    T = B * S
    E = gate_kernel.shape[1]
    H = Wk.shape[2]
    HB = 512 if H % 512 == 0 else H
    NHB = H // HB

    x2 = x.reshape(T, D)
    xb = x2.astype(jnp.bfloat16)
    bk2 = bk.reshape(E, 1, H)
    bv2 = bv.reshape(E, 1, D)
    bks2 = bk_s.reshape(1, H)
    bvs2 = bv_s.reshape(1, D)

    out = pl.pallas_call(
        _moe_body,
        grid=(E, NHB),
        in_specs=[
            pl.BlockSpec((T, D), lambda e, h: (0, 0)),
            pl.BlockSpec((T, D), lambda e, h: (0, 0)),
            pl.BlockSpec((D, E), lambda e, h: (0, 0)),
            pl.BlockSpec((1, D, HB), lambda e, h: (e, 0, h)),
            pl.BlockSpec((1, 1, HB), lambda e, h: (e, 0, h)),
            pl.BlockSpec((1, HB, D), lambda e, h: (e, h, 0)),
            pl.BlockSpec((1, 1, D), lambda e, h: (e, 0, 0)),
            pl.BlockSpec((D, HB), lambda e, h: (0, h)),
            pl.BlockSpec((1, HB), lambda e, h: (0, h)),
            pl.BlockSpec((HB, D), lambda e, h: (h, 0)),
            pl.BlockSpec((1, D), lambda e, h: (0, 0)),
        ],
        out_specs=pl.BlockSpec((T, D), lambda e, h: (0, 0)),
        out_shape=jax.ShapeDtypeStruct((T, D), jnp.float32),
        scratch_shapes=[pltpu.VMEM((T, E), jnp.float32)],
        compiler_params=pltpu.CompilerParams(
            dimension_semantics=("arbitrary", "arbitrary")),
    )(x2, xb, gate_kernel, Wk, bk2, Wv, bv2, Wk_s, bks2, Wv_s, bvs2)

    return (out.reshape(B, S, D), jnp.float32(0.0))
